# baseline (speedup 1.0000x reference)
import jax
import jax.numpy as jnp
from jax import lax
from jax.experimental import pallas as pl
from jax.experimental.pallas import tpu as pltpu
from jax.experimental.pallas import tpu_sc as plsc

_NUM_FACTORS = 32
_NUM_ROWS = 1000000
_BATCH = 16384
_NUM_CORES = 2
_NUM_SUBCORES = 16
_NW = _NUM_CORES * _NUM_SUBCORES
_RPW = _BATCH // _NW
_CHUNK = 64
_NCH = _RPW // _CHUNK
_LANES = 16


def _mf_body(user_id, item_id, P, Q, ub, ib, out,
             uidx, iidx, pu, qi, bu, bi, outv,
             sem0, sem1):
    cid = lax.axis_index("c")
    sid = lax.axis_index("s")
    wid = sid * _NUM_CORES + cid
    base = wid * _RPW

    pltpu.sync_copy(user_id.at[pl.ds(base, _RPW)], uidx)
    pltpu.sync_copy(item_id.at[pl.ds(base, _RPW)], iidx)

    sems = [sem0, sem1]
    bufs = [(pu.at[0], qi.at[0], bu.at[0], bi.at[0]),
            (pu.at[1], qi.at[1], bu.at[1], bi.at[1])]

    def row_copies(b, k, u, i):
        pub, qib, bub, bib = bufs[b]
        s = sems[b]
        dst = pl.ds(k, 1)
        return [
            pltpu.make_async_copy(P.at[pl.ds(u, 1), :], pub.at[dst, :], s),
            pltpu.make_async_copy(Q.at[pl.ds(i, 1), :], qib.at[dst, :], s),
            pltpu.make_async_copy(ub.at[pl.ds(u, 1), :], bub.at[dst, :], s),
            pltpu.make_async_copy(ib.at[pl.ds(i, 1), :], bib.at[dst, :], s),
        ]

    def fire(j, b):
        o = j * _CHUNK

        def one(g2, carry):
            c = g2 * _LANES
            u16 = uidx[pl.ds(o + c, _LANES)]
            i16 = iidx[pl.ds(o + c, _LANES)]
            for kk in range(_LANES):
                for d in row_copies(b, c + kk, u16[kk], i16[kk]):
                    d.start()
            return carry
        lax.fori_loop(0, _CHUNK // _LANES, one, 0)

    def drain(j, b):
        def one(k, carry):
            for d in row_copies(b, k, 0, 0):
                d.wait()
            return carry
        lax.fori_loop(0, _CHUNK, one, 0)

    fire(0, 0)
    fire(1, 1)

    lane_iota = lax.iota(jnp.int32, _LANES)

    def round_body(g, carry):
        for b in range(2):
            j = 2 * g + b
            drain(j, b)
            pub, qib, bub, bib = bufs[b]

            def group(g2, carry2):
                c = g2 * _LANES
                rows = lane_iota + c
                acc = (plsc.load_gather(bub, [rows, jnp.zeros_like(rows)])
                       + plsc.load_gather(bib, [rows, jnp.zeros_like(rows)]))
                for f in range(_NUM_FACTORS):
                    col = jnp.full_like(rows, f)
                    acc = acc + (plsc.load_gather(pub, [rows, col])
                                 * plsc.load_gather(qib, [rows, col]))
                outv[pl.ds(j * _CHUNK + c, _LANES)] = acc
                return carry2

            lax.fori_loop(0, _CHUNK // _LANES, group, 0)
            pl.when(g < _NCH // 2 - 1)(lambda: fire(j + 2, b))
        return carry
    lax.fori_loop(0, _NCH // 2, round_body, 0)

    pltpu.sync_copy(outv, out.at[pl.ds(base, _RPW)])


@jax.jit
def _mf(user_id, item_id, P, Q, ub, ib):
    mesh = plsc.VectorSubcoreMesh(core_axis_name="c", subcore_axis_name="s")
    kern = pl.kernel(
        _mf_body,
        out_type=jax.ShapeDtypeStruct((_BATCH,), jnp.float32),
        mesh=mesh,
        compiler_params=pltpu.CompilerParams(needs_layout_passes=False),
        scratch_types=[
            pltpu.VMEM((_RPW,), jnp.int32),
            pltpu.VMEM((_RPW,), jnp.int32),
            pltpu.VMEM((2, _CHUNK, _NUM_FACTORS), jnp.float32),
            pltpu.VMEM((2, _CHUNK, _NUM_FACTORS), jnp.float32),
            pltpu.VMEM((2, _CHUNK, 1), jnp.float32),
            pltpu.VMEM((2, _CHUNK, 1), jnp.float32),
            pltpu.VMEM((_RPW,), jnp.float32),
            pltpu.SemaphoreType.DMA,
            pltpu.SemaphoreType.DMA,
        ],
    )
    return kern(user_id, item_id, P, Q, ub, ib)


def kernel(user_id, item_id, P, Q, user_bias, item_bias):
    return _mf(user_id, item_id, P, Q, user_bias, item_bias)
